# baseline (speedup 1.0000x reference)
import functools
import math

import jax
import jax.numpy as jnp
from jax import lax
from jax.experimental import pallas as pl
from jax.experimental.pallas import tpu as pltpu
from jax.experimental.pallas import tpu_sc as plsc

D_MODEL = 128
LANES = 16
NUM_CORES = 2
NUM_SUBCORES = 16
NUM_WORKERS = NUM_CORES * NUM_SUBCORES
SCALE = math.sqrt(D_MODEL)


@functools.partial(jax.jit, static_argnames=("b0", "b1"))
def _lookup(idx, table, b0, b1):
    mesh = plsc.VectorSubcoreMesh(core_axis_name="c", subcore_axis_name="s")
    xr = b0 // NUM_WORKERS
    nchunks = xr // 2
    cw = idx.shape[2]
    nbuf = next(d for d in (4, 3, 2, 1) if nchunks % d == 0)
    lead = min(3, nbuf - 1)

    @functools.partial(
        pl.kernel,
        mesh=mesh,
        out_type=jax.ShapeDtypeStruct((b0, b1, D_MODEL), jnp.float32),
        scratch_types=[
            pltpu.VMEM((nchunks, cw), jnp.int32),
            pltpu.VMEM((nbuf, cw, D_MODEL), jnp.float32),
            pltpu.SemaphoreType.DMA((nbuf,)),
            pltpu.SemaphoreType.DMA((nbuf,)),
        ],
    )
    def k(idx_hbm, table_hbm, out_hbm, idx_v, rows_v, gsem, osem):
        cid = lax.axis_index("c")
        sid = lax.axis_index("s")
        wid = sid * NUM_CORES + cid
        x_base = wid * xr
        pltpu.sync_copy(idx_hbm.at[wid], idx_v)

        def start_gather(j, b):
            pltpu.async_copy(table_hbm.at[idx_v.at[j]], rows_v.at[b], gsem.at[b])

        def wait_gather(j, b):
            pltpu.make_async_copy(
                table_hbm.at[idx_v.at[j]], rows_v.at[b], gsem.at[b]
            ).wait()

        def start_scatter(j, b):
            x0 = x_base + j * 2
            pltpu.async_copy(rows_v.at[b, pl.ds(0, b1)], out_hbm.at[x0], osem.at[b])
            pltpu.async_copy(
                rows_v.at[b, pl.ds(b1, b1)], out_hbm.at[x0 + 1], osem.at[b]
            )

        def wait_scatter(b):
            pltpu.make_async_copy(
                rows_v.at[b, pl.ds(0, b1)], out_hbm.at[0], osem.at[b]
            ).wait()
            pltpu.make_async_copy(
                rows_v.at[b, pl.ds(b1, b1)], out_hbm.at[0], osem.at[b]
            ).wait()

        for j in range(lead):
            start_gather(j, j)

        def outer(j0, carry):
            for db in range(nbuf):
                j = j0 + db
                bb = (db + lead) % nbuf

                @pl.when(jnp.logical_and(j + lead < nchunks, j + lead >= nbuf))
                def _():
                    wait_scatter(bb)

                @pl.when(j + lead < nchunks)
                def _():
                    start_gather(j + lead, bb)

                wait_gather(j, db)

                def row_body(r, c2):
                    for c in range(D_MODEL // LANES):
                        sl = pl.ds(c * LANES, LANES)
                        rows_v[db, r, sl] = rows_v[db, r, sl] * SCALE
                    return c2

                lax.fori_loop(0, 2 * b1, row_body, 0, unroll=2)
                start_scatter(j, db)
            return carry

        lax.fori_loop(0, nchunks // nbuf, lambda i, c: outer(i * nbuf, c), 0)

        for b in range(nbuf):
            wait_scatter(b)

    return k(idx, table)


def kernel(x, table):
    b0, b1 = x.shape
    idx = x.astype(jnp.int32).reshape(b0 // 2, 2 * b1)
    cw = -(-(2 * b1) // 8) * 8
    if cw != 2 * b1:
        idx = jnp.pad(idx, ((0, 0), (0, cw - 2 * b1)))
    idx = idx.reshape(NUM_WORKERS, b0 // (2 * NUM_WORKERS), cw)
    return _lookup(idx, table, b0, b1)


# speedup vs baseline: 1.0014x; 1.0014x over previous
import functools
import math

import jax
import jax.numpy as jnp
from jax import lax
from jax.experimental import pallas as pl
from jax.experimental.pallas import tpu as pltpu
from jax.experimental.pallas import tpu_sc as plsc

D_MODEL = 128
LANES = 16
NUM_CORES = 2
NUM_SUBCORES = 16
NUM_WORKERS = NUM_CORES * NUM_SUBCORES
SCALE = math.sqrt(D_MODEL)


@functools.partial(jax.jit, static_argnames=("b0", "b1"))
def _lookup(idx, table, b0, b1):
    mesh = plsc.VectorSubcoreMesh(core_axis_name="c", subcore_axis_name="s")
    xr = b0 // NUM_WORKERS
    nchunks = xr // 2
    cw = idx.shape[2]
    nbuf = next(d for d in (8, 4, 3, 2, 1) if nchunks % d == 0)
    lead = min(5, nbuf - 1)

    @functools.partial(
        pl.kernel,
        mesh=mesh,
        out_type=jax.ShapeDtypeStruct((b0, b1, D_MODEL), jnp.float32),
        scratch_types=[
            pltpu.VMEM((nchunks, cw), jnp.int32),
            pltpu.VMEM((nbuf, cw, D_MODEL), jnp.float32),
            pltpu.SemaphoreType.DMA((nbuf,)),
            pltpu.SemaphoreType.DMA((nbuf,)),
        ],
    )
    def k(idx_hbm, table_hbm, out_hbm, idx_v, rows_v, gsem, osem):
        cid = lax.axis_index("c")
        sid = lax.axis_index("s")
        wid = sid * NUM_CORES + cid
        x_base = wid * xr
        pltpu.sync_copy(idx_hbm.at[wid], idx_v)

        def start_gather(j, b):
            pltpu.async_copy(table_hbm.at[idx_v.at[j]], rows_v.at[b], gsem.at[b])

        def wait_gather(j, b):
            pltpu.make_async_copy(
                table_hbm.at[idx_v.at[j]], rows_v.at[b], gsem.at[b]
            ).wait()

        def start_scatter(j, b):
            x0 = x_base + j * 2
            pltpu.async_copy(rows_v.at[b, pl.ds(0, b1)], out_hbm.at[x0], osem.at[b])
            pltpu.async_copy(
                rows_v.at[b, pl.ds(b1, b1)], out_hbm.at[x0 + 1], osem.at[b]
            )

        def wait_scatter(b):
            pltpu.make_async_copy(
                rows_v.at[b, pl.ds(0, b1)], out_hbm.at[0], osem.at[b]
            ).wait()
            pltpu.make_async_copy(
                rows_v.at[b, pl.ds(b1, b1)], out_hbm.at[0], osem.at[b]
            ).wait()

        for j in range(lead):
            start_gather(j, j)

        def outer(j0, carry):
            for db in range(nbuf):
                j = j0 + db
                bb = (db + lead) % nbuf

                @pl.when(jnp.logical_and(j + lead < nchunks, j + lead >= nbuf))
                def _():
                    wait_scatter(bb)

                @pl.when(j + lead < nchunks)
                def _():
                    start_gather(j + lead, bb)

                wait_gather(j, db)

                def row_body(r, c2):
                    for c in range(D_MODEL // LANES):
                        sl = pl.ds(c * LANES, LANES)
                        rows_v[db, r, sl] = rows_v[db, r, sl] * SCALE
                    return c2

                lax.fori_loop(0, 2 * b1, row_body, 0, unroll=2)
                start_scatter(j, db)
            return carry

        lax.fori_loop(0, nchunks // nbuf, lambda i, c: outer(i * nbuf, c), 0)

        for b in range(nbuf):
            wait_scatter(b)

    return k(idx, table)


def kernel(x, table):
    b0, b1 = x.shape
    idx = x.astype(jnp.int32).reshape(b0 // 2, 2 * b1)
    cw = -(-(2 * b1) // 8) * 8
    if cw != 2 * b1:
        idx = jnp.pad(idx, ((0, 0), (0, cw - 2 * b1)))
    idx = idx.reshape(NUM_WORKERS, b0 // (2 * NUM_WORKERS), cw)
    return _lookup(idx, table, b0, b1)
